# baseline (speedup 1.0000x reference)
import functools
import jax
import jax.numpy as jnp
from jax import lax
from jax.experimental import pallas as pl
from jax.experimental.pallas import tpu as pltpu
from jax.experimental.pallas import tpu_sc as plsc

_DIM = 128


def _sc_lookup(batch, total_len, seg_bounds, dtype):
    info = plsc.get_sparse_core_info()
    nc, ns = info.num_cores, info.num_subcores
    nw = nc * ns
    per_w = batch // nw
    pad_len = ((total_len + 15) // 16) * 16
    mesh = plsc.VectorSubcoreMesh(core_axis_name="c", subcore_axis_name="s")

    @functools.partial(
        pl.kernel,
        mesh=mesh,
        out_type=jax.ShapeDtypeStruct((batch, total_len, _DIM), dtype),
        scratch_types=[
            pltpu.VMEM((pad_len,), jnp.int32),
            pltpu.VMEM((total_len, _DIM), dtype),
            pltpu.SemaphoreType.DMA,
        ],
    )
    def k(emb_hbm, out_hbm, idx_v, pat_v, sem):
        wid = lax.axis_index("s") * nc + lax.axis_index("c")
        base = wid * per_w
        for c in range(pad_len // 16):
            row = lax.iota(jnp.int32, 16) + (c * 16)
            t = jnp.full((16,), seg_bounds[-1][1], dtype=jnp.int32)
            for hi, ty in reversed(seg_bounds[:-1]):
                t = jnp.where(row < hi, jnp.full((16,), ty, jnp.int32), t)
            idx_v[pl.ds(c * 16, 16)] = t
        g1 = pltpu.async_copy(
            emb_hbm.at[idx_v.at[pl.ds(0, 128)]], pat_v.at[pl.ds(0, 128)], sem)
        g2 = pltpu.async_copy(
            emb_hbm.at[idx_v.at[pl.ds(128, total_len - 128)]],
            pat_v.at[pl.ds(128, total_len - 128)], sem)
        g1.wait()
        g2.wait()
        copies = [
            pltpu.async_copy(pat_v, out_hbm.at[base + j], sem)
            for j in range(per_w)
        ]
        for cp in copies:
            cp.wait()

    return k


def kernel(ego, obs, lane, bound, embedding):
    batch = ego.shape[0]
    obs_len = obs.shape[1]
    lanes_len = lane.shape[1]
    bounds_len = bound.shape[1]
    total_len = 1 + obs_len + 1 + lanes_len + bounds_len
    route_start = 1 + obs_len
    lanes_start = route_start + 1
    bounds_start = lanes_start + lanes_len
    seg_bounds = [
        (1, 0),
        (route_start, 2),
        (lanes_start, 3),
        (bounds_start, 4),
        (None, 5),
    ]
    return _sc_lookup(batch, total_len, seg_bounds, embedding.dtype)(embedding)
